# baseline (speedup 1.0000x reference)
import functools

import jax
import jax.numpy as jnp
from jax import lax
from jax.experimental import pallas as pl
from jax.experimental.pallas import tpu as pltpu
from jax.experimental.pallas import tpu_sc as plsc

D_MODEL = 512
TABLE_ROWS = 10000
P1 = 73856093 % TABLE_ROWS
P2 = 19349663 % TABLE_ROWS
P3 = 83492791 % TABLE_ROWS
CHUNK = 32
NBUF = 7
DEPTH = 5
LANES = 16


def _build_sc_call(total, nw):
    b_per_w = total // nw
    nch = b_per_w // CHUNK
    assert b_per_w % CHUNK == 0 and nch >= 2 * NBUF
    rounds = -(-nch // NBUF)
    grp = b_per_w // LANES
    mesh = plsc.VectorSubcoreMesh(core_axis_name="c", subcore_axis_name="s")
    nc = mesh.num_cores

    @functools.partial(
        pl.kernel,
        out_type=jax.ShapeDtypeStruct((total, D_MODEL), jnp.float32),
        mesh=mesh,
        scratch_types=[
            pltpu.VMEM((b_per_w,), jnp.float32),
            pltpu.VMEM((b_per_w,), jnp.float32),
            pltpu.VMEM((b_per_w,), jnp.float32),
            pltpu.VMEM((b_per_w,), jnp.int32),
            pltpu.VMEM((NBUF, CHUNK, D_MODEL), jnp.float32),
            *([pltpu.SemaphoreType.DMA] * (2 * NBUF)),
        ],
    )
    def sc_kernel(x_hbm, y_hbm, z_hbm, pe_hbm, out_hbm, x_v, y_v, z_v, idx_v, rows_v,
                  *sems):
        gsems = sems[:NBUF]
        wsems = sems[NBUF:]
        wid = lax.axis_index("s") * nc + lax.axis_index("c")
        base_pt = wid * b_per_w

        cp_x = pltpu.async_copy(x_hbm.at[pl.ds(base_pt, b_per_w)], x_v, sems[0])
        cp_y = pltpu.async_copy(y_hbm.at[pl.ds(base_pt, b_per_w)], y_v, sems[1])
        cp_z = pltpu.async_copy(z_hbm.at[pl.ds(base_pt, b_per_w)], z_v, sems[2])
        cp_x.wait()
        cp_y.wait()
        cp_z.wait()

        m_i32 = jnp.int32(TABLE_ROWS)
        comps = (x_v, y_v, z_v)

        def hash_chunk(t):
            pt0 = t * jnp.int32(CHUNK)
            for g in range(CHUNK // LANES):
                pt = pt0 + jnp.int32(g * LANES)

                def term(comp, mult):
                    v = comps[comp][pl.ds(pt, LANES)]
                    a = (v * 1000.0).astype(jnp.int32)
                    r = lax.rem(a, m_i32) + m_i32
                    return r * jnp.int32(mult)

                h = term(0, P1) + term(1, P2) + term(2, P3)
                idx_v[pl.ds(pt, LANES)] = lax.rem(h, m_i32)

        def start_g(t, b):
            pltpu.async_copy(
                pe_hbm.at[idx_v.at[pl.ds(t * CHUNK, CHUNK)]],
                rows_v.at[jnp.int32(b)],
                gsems[b],
            )

        def wait_g(b):
            pltpu.make_async_copy(
                pe_hbm.at[idx_v.at[pl.ds(0, CHUNK)]], rows_v.at[jnp.int32(b)], gsems[b]
            ).wait()

        def start_w(t, b):
            pltpu.async_copy(
                rows_v.at[jnp.int32(b)],
                out_hbm.at[pl.ds(base_pt + t * CHUNK, CHUNK)],
                wsems[b],
            )

        def wait_w(b):
            pltpu.make_async_copy(
                rows_v.at[jnp.int32(b)], out_hbm.at[pl.ds(0, CHUNK)], wsems[b]
            ).wait()

        for t in range(DEPTH):
            hash_chunk(jnp.int32(t))
            start_g(t, t)

        def pipe_body(_, t0):
            for b in range(NBUF):
                t = t0 + jnp.int32(b)

                @pl.when(t < nch)
                def _():
                    wait_g(b)
                    start_w(t, b)

                bd = (b + DEPTH) % NBUF
                td = t + jnp.int32(DEPTH)

                @pl.when((t >= NBUF - DEPTH) & (td < nch))
                def _():
                    wait_w(bd)

                @pl.when(td < nch)
                def _():
                    hash_chunk(td)
                    start_g(td, bd)

            return t0 + jnp.int32(NBUF)

        lax.fori_loop(0, rounds, pipe_body, jnp.int32(0))
        for b in range(NBUF):
            wait_w(b)

    return sc_kernel


def kernel(xyz, pe):
    b, n, _ = xyz.shape
    total = b * n
    info = plsc.get_sparse_core_info()
    nw = info.num_cores * info.num_subcores
    sc_call = _build_sc_call(total, nw)
    flat = xyz.reshape(total, 3)
    out = sc_call(flat[:, 0], flat[:, 1], flat[:, 2], pe)
    return out.reshape(b, n, D_MODEL)


# speedup vs baseline: 1.0097x; 1.0097x over previous
import functools

import jax
import jax.numpy as jnp
from jax import lax
from jax.experimental import pallas as pl
from jax.experimental.pallas import tpu as pltpu
from jax.experimental.pallas import tpu_sc as plsc

D_MODEL = 512
TABLE_ROWS = 10000
P1 = 73856093 % TABLE_ROWS
P2 = 19349663 % TABLE_ROWS
P3 = 83492791 % TABLE_ROWS
CHUNK = 32
NBUF = 6
DEPTH = 4
LANES = 16


def _build_sc_call(total, nw):
    b_per_w = total // nw
    nch = b_per_w // CHUNK
    assert b_per_w % CHUNK == 0 and nch >= 2 * NBUF
    rounds = -(-nch // NBUF)
    mesh = plsc.VectorSubcoreMesh(core_axis_name="c", subcore_axis_name="s")
    nc = mesh.num_cores

    @functools.partial(
        pl.kernel,
        out_type=jax.ShapeDtypeStruct((total, D_MODEL), jnp.float32),
        mesh=mesh,
        scratch_types=[
            pltpu.VMEM((b_per_w,), jnp.float32),
            pltpu.VMEM((b_per_w,), jnp.float32),
            pltpu.VMEM((b_per_w,), jnp.float32),
            pltpu.VMEM((b_per_w,), jnp.int32),
            pltpu.VMEM((NBUF, CHUNK, D_MODEL), jnp.float32),
            *([pltpu.SemaphoreType.DMA] * (2 * NBUF)),
        ],
    )
    def sc_kernel(x_hbm, y_hbm, z_hbm, pe_hbm, out_hbm, x_v, y_v, z_v, idx_v, rows_v,
                  *sems):
        gsems = sems[:NBUF]
        wsems = sems[NBUF:]
        wid = lax.axis_index("s") * nc + lax.axis_index("c")
        base_pt = wid * b_per_w

        cp_x = pltpu.async_copy(x_hbm.at[pl.ds(base_pt, b_per_w)], x_v, sems[0])
        cp_y = pltpu.async_copy(y_hbm.at[pl.ds(base_pt, b_per_w)], y_v, sems[1])
        cp_z = pltpu.async_copy(z_hbm.at[pl.ds(base_pt, b_per_w)], z_v, sems[2])
        cp_x.wait()
        cp_y.wait()
        cp_z.wait()

        m_i32 = jnp.int32(TABLE_ROWS)
        comps = (x_v, y_v, z_v)

        def hash_chunk(t):
            pt0 = t * jnp.int32(CHUNK)
            for g in range(CHUNK // LANES):
                pt = pt0 + jnp.int32(g * LANES)

                def term(comp, mult):
                    v = comps[comp][pl.ds(pt, LANES)]
                    a = (v * 1000.0).astype(jnp.int32)
                    r = lax.rem(a, m_i32) + m_i32
                    return r * jnp.int32(mult)

                h = term(0, P1) + term(1, P2) + term(2, P3)
                idx_v[pl.ds(pt, LANES)] = lax.rem(h, m_i32)

        def start_g(t, b):
            pltpu.async_copy(
                pe_hbm.at[idx_v.at[pl.ds(t * CHUNK, CHUNK)]],
                rows_v.at[jnp.int32(b)],
                gsems[b],
            )

        def wait_g(b):
            pltpu.make_async_copy(
                pe_hbm.at[idx_v.at[pl.ds(0, CHUNK)]], rows_v.at[jnp.int32(b)], gsems[b]
            ).wait()

        def start_w(t, b):
            pltpu.async_copy(
                rows_v.at[jnp.int32(b)],
                out_hbm.at[pl.ds(base_pt + t * CHUNK, CHUNK)],
                wsems[b],
            )

        def wait_w(b):
            pltpu.make_async_copy(
                rows_v.at[jnp.int32(b)], out_hbm.at[pl.ds(0, CHUNK)], wsems[b]
            ).wait()

        for t in range(DEPTH):
            hash_chunk(jnp.int32(t))
            start_g(t, t)

        def pipe_body(_, t0):
            for b in range(NBUF):
                t = t0 + jnp.int32(b)

                @pl.when(t < nch)
                def _():
                    wait_g(b)
                    start_w(t, b)

                bd = (b + DEPTH) % NBUF
                td = t + jnp.int32(DEPTH)

                @pl.when((t >= NBUF - DEPTH) & (td < nch))
                def _():
                    wait_w(bd)

                @pl.when(td < nch)
                def _():
                    hash_chunk(td)
                    start_g(td, bd)

            return t0 + jnp.int32(NBUF)

        lax.fori_loop(0, rounds, pipe_body, jnp.int32(0))
        for b in range(NBUF):
            wait_w(b)

    return sc_kernel


def kernel(xyz, pe):
    b, n, _ = xyz.shape
    total = b * n
    info = plsc.get_sparse_core_info()
    nw = info.num_cores * info.num_subcores
    sc_call = _build_sc_call(total, nw)
    flat = xyz.reshape(total, 3)
    out = sc_call(flat[:, 0], flat[:, 1], flat[:, 2], pe)
    return out.reshape(b, n, D_MODEL)
